# baseline (speedup 1.0000x reference)
import jax
import jax.numpy as jnp
from jax.experimental import pallas as pl
from jax.experimental.pallas import tpu as pltpu

_T = 4
_N = 4096
_IN_DIM = 512
_HID = 256
_NCLS = 32
_K = 4
_C = _T * _NCLS
_BLK = 256
_SB = 2 * _BLK
_NB = _N // _SB
_CH = 2048


def _gpr_body(x_ref, a0_ref, a1_ref, w1_ref, w2_ref, b1_ref, b2_ref, g_ref,
              z_ref, a_scr, acc_scr, hb0_scr, hb1_scr):
    j = pl.program_id(0)
    rows = pl.ds(j * _SB, _SB)

    a_scr[:, pl.ds(j * _SB, _BLK)] = a0_ref[...].astype(jnp.bfloat16)
    a_scr[:, pl.ds(j * _SB + _BLK, _BLK)] = a1_ref[...].astype(jnp.bfloat16)

    h1 = jnp.dot(x_ref[...].astype(jnp.bfloat16), w1_ref[...],
                 preferred_element_type=jnp.float32)
    h1 = jnp.maximum(h1 + b1_ref[...], 0.0).astype(jnp.bfloat16)
    h0 = jnp.dot(h1, w2_ref[...], preferred_element_type=jnp.float32)
    h0 = h0 + b2_ref[...]
    z_ref[rows, :] = g_ref[0][None, :] * h0

    part = jnp.dot(a_scr[:, rows], h0.astype(jnp.bfloat16),
                   preferred_element_type=jnp.float32)

    @pl.when(j == 0)
    def _init():
        acc_scr[...] = part

    @pl.when(j > 0)
    def _accum():
        acc_scr[...] += part

    @pl.when(j == _NB - 1)
    def _tail():
        bufs = [hb0_scr, hb1_scr]
        for c in range(_N // _CH):
            ch = pl.ds(c * _CH, _CH)
            h1f = acc_scr[ch, :]
            z_ref[ch, :] += g_ref[1][None, :] * h1f
            hb0_scr[ch, :] = h1f.astype(jnp.bfloat16)
        for k in range(2, _K + 1):
            src = bufs[k % 2]
            dst = bufs[(k + 1) % 2]
            h = src[...]
            for c in range(_N // _CH):
                ch = pl.ds(c * _CH, _CH)
                hn = jnp.dot(a_scr[ch, :], h, preferred_element_type=jnp.float32)
                z_ref[ch, :] += g_ref[k][None, :] * hn
                if k < _K:
                    dst[ch, :] = hn.astype(jnp.bfloat16)


def kernel(x, A_hat, W1, b1, W2, b2, gamma):
    w1c = W1.transpose(1, 0, 2).reshape(_IN_DIM, _T * _HID).astype(jnp.bfloat16)
    w2bd = jax.scipy.linalg.block_diag(*[W2[t] for t in range(_T)]).astype(jnp.bfloat16)
    b1c = b1.reshape(1, _T * _HID)
    b2c = b2.reshape(1, _C)
    gexp = jnp.repeat(gamma.T, _NCLS, axis=1)
    gexp = jnp.zeros((8, _C), jnp.float32).at[: _K + 1].set(gexp)

    zflat = pl.pallas_call(
        _gpr_body,
        grid=(_NB,),
        in_specs=[
            pl.BlockSpec((_SB, _IN_DIM), lambda j: (j, 0)),
            pl.BlockSpec((_N, _BLK), lambda j: (0, 2 * j)),
            pl.BlockSpec((_N, _BLK), lambda j: (0, 2 * j + 1)),
            pl.BlockSpec((_IN_DIM, _T * _HID), lambda j: (0, 0)),
            pl.BlockSpec((_T * _HID, _C), lambda j: (0, 0)),
            pl.BlockSpec((1, _T * _HID), lambda j: (0, 0)),
            pl.BlockSpec((1, _C), lambda j: (0, 0)),
            pl.BlockSpec((8, _C), lambda j: (0, 0)),
        ],
        out_specs=pl.BlockSpec((_N, _C), lambda j: (0, 0)),
        out_shape=jax.ShapeDtypeStruct((_N, _C), jnp.float32),
        scratch_shapes=[
            pltpu.VMEM((_N, _N), jnp.bfloat16),
            pltpu.VMEM((_N, _C), jnp.float32),
            pltpu.VMEM((_N, _C), jnp.bfloat16),
            pltpu.VMEM((_N, _C), jnp.bfloat16),
        ],
        compiler_params=pltpu.CompilerParams(
            vmem_limit_bytes=62 * 1024 * 1024,
        ),
    )(x, A_hat, A_hat, w1c, w2bd, b1c, b2c, gexp)
    return zflat.reshape(_N, _T, _NCLS).transpose(1, 0, 2)


# speedup vs baseline: 1.0072x; 1.0072x over previous
import jax
import jax.numpy as jnp
from jax.experimental import pallas as pl
from jax.experimental.pallas import tpu as pltpu

_T = 4
_N = 4096
_IN_DIM = 512
_HID = 256
_NCLS = 32
_K = 4
_C = _T * _NCLS
_BLK = 256
_NB = _N // _BLK
_CH = 2048


def _gpr_body(x_ref, a_ref, w1_ref, w2_ref, b1_ref, b2_ref, g_ref, z_ref,
              a_scr, acc_scr, hb0_scr, hb1_scr):
    j = pl.program_id(0)
    rows = pl.ds(j * _BLK, _BLK)

    a_scr[:, rows] = a_ref[...].astype(jnp.bfloat16)

    h1 = jnp.dot(x_ref[...].astype(jnp.bfloat16), w1_ref[...],
                 preferred_element_type=jnp.float32)
    h1 = jnp.maximum(h1 + b1_ref[...], 0.0).astype(jnp.bfloat16)
    h0 = jnp.dot(h1, w2_ref[...], preferred_element_type=jnp.float32)
    h0 = h0 + b2_ref[...]
    z_ref[rows, :] = g_ref[0][None, :] * h0

    part = jnp.dot(a_scr[:, rows], h0.astype(jnp.bfloat16),
                   preferred_element_type=jnp.float32)

    @pl.when(j == 0)
    def _init():
        acc_scr[...] = part

    @pl.when(j > 0)
    def _accum():
        acc_scr[...] += part

    @pl.when(j == _NB - 1)
    def _tail():
        bufs = [hb0_scr, hb1_scr]
        for c in range(_N // _CH):
            ch = pl.ds(c * _CH, _CH)
            h1f = acc_scr[ch, :]
            z_ref[ch, :] += g_ref[1][None, :] * h1f
            hb0_scr[ch, :] = h1f.astype(jnp.bfloat16)
        for k in range(2, _K + 1):
            src = bufs[k % 2]
            dst = bufs[(k + 1) % 2]
            h = src[...]
            for c in range(_N // _CH):
                ch = pl.ds(c * _CH, _CH)
                hn = jnp.dot(a_scr[ch, :], h, preferred_element_type=jnp.float32)
                z_ref[ch, :] += g_ref[k][None, :] * hn
                if k < _K:
                    dst[ch, :] = hn.astype(jnp.bfloat16)


def kernel(x, A_hat, W1, b1, W2, b2, gamma):
    w1c = W1.transpose(1, 0, 2).reshape(_IN_DIM, _T * _HID).astype(jnp.bfloat16)
    w2bd = jax.scipy.linalg.block_diag(*[W2[t] for t in range(_T)]).astype(jnp.bfloat16)
    b1c = b1.reshape(1, _T * _HID)
    b2c = b2.reshape(1, _C)
    gexp = jnp.repeat(gamma.T, _NCLS, axis=1)
    gexp = jnp.zeros((8, _C), jnp.float32).at[: _K + 1].set(gexp)

    zflat = pl.pallas_call(
        _gpr_body,
        grid=(_NB,),
        in_specs=[
            pl.BlockSpec((_BLK, _IN_DIM), lambda j: (j, 0)),
            pl.BlockSpec((_N, _BLK), lambda j: (0, j)),
            pl.BlockSpec((_IN_DIM, _T * _HID), lambda j: (0, 0)),
            pl.BlockSpec((_T * _HID, _C), lambda j: (0, 0)),
            pl.BlockSpec((1, _T * _HID), lambda j: (0, 0)),
            pl.BlockSpec((1, _C), lambda j: (0, 0)),
            pl.BlockSpec((8, _C), lambda j: (0, 0)),
        ],
        out_specs=pl.BlockSpec((_N, _C), lambda j: (0, 0)),
        out_shape=jax.ShapeDtypeStruct((_N, _C), jnp.float32),
        scratch_shapes=[
            pltpu.VMEM((_N, _N), jnp.bfloat16),
            pltpu.VMEM((_N, _C), jnp.float32),
            pltpu.VMEM((_N, _C), jnp.bfloat16),
            pltpu.VMEM((_N, _C), jnp.bfloat16),
        ],
        compiler_params=pltpu.CompilerParams(
            vmem_limit_bytes=60 * 1024 * 1024,
        ),
    )(x, A_hat, w1c, w2bd, b1c, b2c, gexp)
    return zflat.reshape(_N, _T, _NCLS).transpose(1, 0, 2)
